# baseline (speedup 1.0000x reference)
import functools

import jax
import jax.numpy as jnp
from jax import lax
from jax.experimental import pallas as pl
from jax.experimental.pallas import tpu as pltpu
from jax.experimental.pallas import tpu_sc as plsc

N_TOKENS = 16384
D_MODEL = 4096
N_EXPERTS = 64
K_TOP = 8

_CHUNK_SIZES = (7168, 7168, 2048)


_TM = 1024


def _scores_body(x_ref, wt_ref, s_ref):
    logits = lax.dot_general(
        x_ref[...], wt_ref[...],
        dimension_numbers=(((1,), (0,)), ((), ())),
        preferred_element_type=jnp.float32,
    )
    m = jnp.max(logits, axis=1, keepdims=True)
    e = jnp.exp(logits - m)
    s_ref[...] = e / jnp.sum(e, axis=1, keepdims=True)


def _gate_scores(x, wt, row0, n_rows):
    blk0 = row0 // _TM
    return pl.pallas_call(
        _scores_body,
        grid=(n_rows // _TM,),
        in_specs=[
            pl.BlockSpec((_TM, D_MODEL), lambda i: (blk0 + i, 0)),
            pl.BlockSpec((D_MODEL, N_EXPERTS), lambda i: (0, 0)),
        ],
        out_specs=pl.BlockSpec((_TM, N_EXPERTS), lambda i: (i, 0)),
        out_shape=jax.ShapeDtypeStruct((n_rows, N_EXPERTS), jnp.float32),
    )(x, wt)


_NC, _NS, _L = 2, 16, 16
_NW = _NC * _NS
_VCHUNKS = N_EXPERTS // _L


def _topk_body(rows_w, scores_hbm, vals_hbm, idx_hbm, s_v, val_v, idx_v):
    wid = lax.axis_index("s") * _NC + lax.axis_index("c")
    row0 = wid * rows_w
    pltpu.sync_copy(scores_hbm.at[pl.ds(row0 * N_EXPERTS, rows_w * N_EXPERTS)],
                    s_v)

    lanes = lax.iota(jnp.int32, _L)
    low8 = lanes < K_TOP
    idx_of = [lanes + j * _L for j in range(_VCHUNKS)]

    @plsc.parallel_loop(0, rows_w, unroll=4)
    def _(r):
        parts = []
        for j in range(_VCHUNKS):
            kj = s_v[pl.ds(r * N_EXPERTS + j * _L, _L)]
            parts.append(plsc.sort_key_val(kj, idx_of[j], descending=(j % 2 == 0)))
        (k0, v0), (k1, v1), (k2, v2), (k3, v3) = parts
        t01 = plsc.sort_key_val(jnp.where(low8, k0, k1), jnp.where(low8, v0, v1),
                                descending=True)
        t23 = plsc.sort_key_val(jnp.where(low8, k2, k3), jnp.where(low8, v2, v3),
                                descending=False)
        kt, vt = plsc.sort_key_val(jnp.where(low8, t01[0], t23[0]),
                                   jnp.where(low8, t01[1], t23[1]),
                                   descending=True)
        plsc.store_compressed(val_v.at[pl.ds(r * K_TOP, _L)], kt, mask=low8)
        plsc.store_compressed(idx_v.at[pl.ds(r * K_TOP, _L)], vt, mask=low8)

    n_out = rows_w * K_TOP
    pltpu.sync_copy(val_v.at[pl.ds(0, n_out)],
                    vals_hbm.at[pl.ds(row0 * K_TOP, n_out)])
    pltpu.sync_copy(idx_v.at[pl.ds(0, n_out)],
                    idx_hbm.at[pl.ds(row0 * K_TOP, n_out)])


@functools.cache
def _get_topk(n_rows):
    rows_w = n_rows // _NW
    return pl.kernel(
        functools.partial(_topk_body, rows_w),
        out_type=[
            jax.ShapeDtypeStruct((n_rows * K_TOP,), jnp.float32),
            jax.ShapeDtypeStruct((n_rows * K_TOP,), jnp.int32),
        ],
        mesh=plsc.VectorSubcoreMesh(core_axis_name="c", subcore_axis_name="s",
                                    num_cores=_NC, num_subcores=_NS),
        compiler_params=pltpu.CompilerParams(needs_layout_passes=False),
        scratch_types=[
            pltpu.VMEM((rows_w * N_EXPERTS,), jnp.float32),
            pltpu.VMEM((rows_w * K_TOP + _L,), jnp.float32),
            pltpu.VMEM((rows_w * K_TOP + _L,), jnp.int32),
        ],
    )


def kernel(x, W):
    wt = W.T
    vals, idx = [], []
    row0 = 0
    for n_rows in _CHUNK_SIZES:
        scores = _gate_scores(x, wt, row0, n_rows)
        v, i = _get_topk(n_rows)(scores.reshape(-1))
        vals.append(v.reshape(n_rows, K_TOP))
        idx.append(i.reshape(n_rows, K_TOP))
        row0 += n_rows
    return (jnp.concatenate(vals, axis=0), jnp.concatenate(idx, axis=0))


# speedup vs baseline: 1.0320x; 1.0320x over previous
import functools

import jax
import jax.numpy as jnp
from jax import lax
from jax.experimental import pallas as pl
from jax.experimental.pallas import tpu as pltpu
from jax.experimental.pallas import tpu_sc as plsc

N_TOKENS = 16384
D_MODEL = 4096
N_EXPERTS = 64
K_TOP = 8

_CHUNK_SIZES = (10240, 6144)


_TM = 1024


def _scores_body(x_ref, wt_ref, s_ref):
    logits = lax.dot_general(
        x_ref[...], wt_ref[...],
        dimension_numbers=(((1,), (0,)), ((), ())),
        preferred_element_type=jnp.float32,
    )
    m = jnp.max(logits, axis=1, keepdims=True)
    e = jnp.exp(logits - m)
    s_ref[...] = e / jnp.sum(e, axis=1, keepdims=True)


def _gate_scores(x, wt, row0, n_rows):
    blk0 = row0 // _TM
    return pl.pallas_call(
        _scores_body,
        grid=(n_rows // _TM,),
        in_specs=[
            pl.BlockSpec((_TM, D_MODEL), lambda i: (blk0 + i, 0)),
            pl.BlockSpec((D_MODEL, N_EXPERTS), lambda i: (0, 0)),
        ],
        out_specs=pl.BlockSpec((_TM, N_EXPERTS), lambda i: (i, 0)),
        out_shape=jax.ShapeDtypeStruct((n_rows, N_EXPERTS), jnp.float32),
    )(x, wt)


_NC, _NS, _L = 2, 16, 16
_NW = _NC * _NS
_VCHUNKS = N_EXPERTS // _L


def _topk_body(rows_w, scores_hbm, vals_hbm, idx_hbm, s_v, val_v, idx_v):
    wid = lax.axis_index("s") * _NC + lax.axis_index("c")
    row0 = wid * rows_w
    pltpu.sync_copy(scores_hbm.at[pl.ds(row0 * N_EXPERTS, rows_w * N_EXPERTS)],
                    s_v)

    lanes = lax.iota(jnp.int32, _L)
    low8 = lanes < K_TOP
    idx_of = [lanes + j * _L for j in range(_VCHUNKS)]

    @plsc.parallel_loop(0, rows_w, unroll=4)
    def _(r):
        parts = []
        for j in range(_VCHUNKS):
            kj = s_v[pl.ds(r * N_EXPERTS + j * _L, _L)]
            parts.append(plsc.sort_key_val(kj, idx_of[j], descending=(j % 2 == 0)))
        (k0, v0), (k1, v1), (k2, v2), (k3, v3) = parts
        t01 = plsc.sort_key_val(jnp.where(low8, k0, k1), jnp.where(low8, v0, v1),
                                descending=True)
        t23 = plsc.sort_key_val(jnp.where(low8, k2, k3), jnp.where(low8, v2, v3),
                                descending=False)
        kt, vt = plsc.sort_key_val(jnp.where(low8, t01[0], t23[0]),
                                   jnp.where(low8, t01[1], t23[1]),
                                   descending=True)
        plsc.store_compressed(val_v.at[pl.ds(r * K_TOP, _L)], kt, mask=low8)
        plsc.store_compressed(idx_v.at[pl.ds(r * K_TOP, _L)], vt, mask=low8)

    n_out = rows_w * K_TOP
    pltpu.sync_copy(val_v.at[pl.ds(0, n_out)],
                    vals_hbm.at[pl.ds(row0 * K_TOP, n_out)])
    pltpu.sync_copy(idx_v.at[pl.ds(0, n_out)],
                    idx_hbm.at[pl.ds(row0 * K_TOP, n_out)])


@functools.cache
def _get_topk(n_rows):
    rows_w = n_rows // _NW
    return pl.kernel(
        functools.partial(_topk_body, rows_w),
        out_type=[
            jax.ShapeDtypeStruct((n_rows * K_TOP,), jnp.float32),
            jax.ShapeDtypeStruct((n_rows * K_TOP,), jnp.int32),
        ],
        mesh=plsc.VectorSubcoreMesh(core_axis_name="c", subcore_axis_name="s",
                                    num_cores=_NC, num_subcores=_NS),
        compiler_params=pltpu.CompilerParams(needs_layout_passes=False),
        scratch_types=[
            pltpu.VMEM((rows_w * N_EXPERTS,), jnp.float32),
            pltpu.VMEM((rows_w * K_TOP + _L,), jnp.float32),
            pltpu.VMEM((rows_w * K_TOP + _L,), jnp.int32),
        ],
    )


def kernel(x, W):
    wt = W.T
    vals, idx = [], []
    row0 = 0
    for n_rows in _CHUNK_SIZES:
        scores = _gate_scores(x, wt, row0, n_rows)
        v, i = _get_topk(n_rows)(scores.reshape(-1))
        vals.append(v.reshape(n_rows, K_TOP))
        idx.append(i.reshape(n_rows, K_TOP))
        row0 += n_rows
    return (jnp.concatenate(vals, axis=0), jnp.concatenate(idx, axis=0))


# speedup vs baseline: 1.0542x; 1.0215x over previous
import functools

import jax
import jax.numpy as jnp
from jax import lax
from jax.experimental import pallas as pl
from jax.experimental.pallas import tpu as pltpu
from jax.experimental.pallas import tpu_sc as plsc

N_TOKENS = 16384
D_MODEL = 4096
N_EXPERTS = 64
K_TOP = 8

_CHUNK_SIZES = (8192, 8192)


_TM = 1024


def _scores_body(x_ref, wt_ref, s_ref):
    logits = lax.dot_general(
        x_ref[...], wt_ref[...],
        dimension_numbers=(((1,), (0,)), ((), ())),
        preferred_element_type=jnp.float32,
    )
    m = jnp.max(logits, axis=1, keepdims=True)
    e = jnp.exp(logits - m)
    s_ref[...] = e / jnp.sum(e, axis=1, keepdims=True)


def _gate_scores(x, wt, row0, n_rows):
    blk0 = row0 // _TM
    return pl.pallas_call(
        _scores_body,
        grid=(n_rows // _TM,),
        in_specs=[
            pl.BlockSpec((_TM, D_MODEL), lambda i: (blk0 + i, 0)),
            pl.BlockSpec((D_MODEL, N_EXPERTS), lambda i: (0, 0)),
        ],
        out_specs=pl.BlockSpec((_TM, N_EXPERTS), lambda i: (i, 0)),
        out_shape=jax.ShapeDtypeStruct((n_rows, N_EXPERTS), jnp.float32),
    )(x, wt)


_NC, _NS, _L = 2, 16, 16
_NW = _NC * _NS
_VCHUNKS = N_EXPERTS // _L


def _topk_body(rows_w, scores_hbm, vals_hbm, idx_hbm, s_v, val_v, idx_v):
    wid = lax.axis_index("s") * _NC + lax.axis_index("c")
    row0 = wid * rows_w
    pltpu.sync_copy(scores_hbm.at[pl.ds(row0 * N_EXPERTS, rows_w * N_EXPERTS)],
                    s_v)

    lanes = lax.iota(jnp.int32, _L)
    low8 = lanes < K_TOP
    idx_of = [lanes + j * _L for j in range(_VCHUNKS)]

    @plsc.parallel_loop(0, rows_w, unroll=4)
    def _(r):
        parts = []
        for j in range(_VCHUNKS):
            kj = s_v[pl.ds(r * N_EXPERTS + j * _L, _L)]
            parts.append(plsc.sort_key_val(kj, idx_of[j], descending=(j % 2 == 0)))
        (k0, v0), (k1, v1), (k2, v2), (k3, v3) = parts
        t01 = plsc.sort_key_val(jnp.where(low8, k0, k1), jnp.where(low8, v0, v1),
                                descending=True)
        t23 = plsc.sort_key_val(jnp.where(low8, k2, k3), jnp.where(low8, v2, v3),
                                descending=False)
        kt, vt = plsc.sort_key_val(jnp.where(low8, t01[0], t23[0]),
                                   jnp.where(low8, t01[1], t23[1]),
                                   descending=True)
        plsc.store_compressed(val_v.at[pl.ds(r * K_TOP, _L)], kt, mask=low8)
        plsc.store_compressed(idx_v.at[pl.ds(r * K_TOP, _L)], vt, mask=low8)

    n_out = rows_w * K_TOP
    pltpu.sync_copy(val_v.at[pl.ds(0, n_out)],
                    vals_hbm.at[pl.ds(row0 * K_TOP, n_out)])
    pltpu.sync_copy(idx_v.at[pl.ds(0, n_out)],
                    idx_hbm.at[pl.ds(row0 * K_TOP, n_out)])


@functools.cache
def _get_topk(n_rows):
    rows_w = n_rows // _NW
    return pl.kernel(
        functools.partial(_topk_body, rows_w),
        out_type=[
            jax.ShapeDtypeStruct((n_rows * K_TOP,), jnp.float32),
            jax.ShapeDtypeStruct((n_rows * K_TOP,), jnp.int32),
        ],
        mesh=plsc.VectorSubcoreMesh(core_axis_name="c", subcore_axis_name="s",
                                    num_cores=_NC, num_subcores=_NS),
        compiler_params=pltpu.CompilerParams(needs_layout_passes=False),
        scratch_types=[
            pltpu.VMEM((rows_w * N_EXPERTS,), jnp.float32),
            pltpu.VMEM((rows_w * K_TOP + _L,), jnp.float32),
            pltpu.VMEM((rows_w * K_TOP + _L,), jnp.int32),
        ],
    )


def kernel(x, W):
    wt = W.T
    vals, idx = [], []
    row0 = 0
    for n_rows in _CHUNK_SIZES:
        scores = _gate_scores(x, wt, row0, n_rows)
        v, i = _get_topk(n_rows)(scores.reshape(-1))
        vals.append(v.reshape(n_rows, K_TOP))
        idx.append(i.reshape(n_rows, K_TOP))
        row0 += n_rows
    return (jnp.concatenate(vals, axis=0), jnp.concatenate(idx, axis=0))


# speedup vs baseline: 1.0862x; 1.0304x over previous
import functools

import jax
import jax.numpy as jnp
from jax import lax
from jax.experimental import pallas as pl
from jax.experimental.pallas import tpu as pltpu
from jax.experimental.pallas import tpu_sc as plsc

N_TOKENS = 16384
D_MODEL = 4096
N_EXPERTS = 64
K_TOP = 8

_CHUNK_SIZES = (8192, 8192)


_TM = 1024


def _scores_body(x_ref, wt_ref, s_ref):
    logits = lax.dot_general(
        x_ref[...], wt_ref[...],
        dimension_numbers=(((1,), (1,)), ((), ())),
        preferred_element_type=jnp.float32,
    )
    m = jnp.max(logits, axis=1, keepdims=True)
    e = jnp.exp(logits - m)
    s_ref[...] = e / jnp.sum(e, axis=1, keepdims=True)


def _gate_scores(x, wt, row0, n_rows):
    blk0 = row0 // _TM
    return pl.pallas_call(
        _scores_body,
        grid=(n_rows // _TM,),
        in_specs=[
            pl.BlockSpec((_TM, D_MODEL), lambda i: (blk0 + i, 0)),
            pl.BlockSpec((N_EXPERTS, D_MODEL), lambda i: (0, 0)),
        ],
        out_specs=pl.BlockSpec((_TM, N_EXPERTS), lambda i: (i, 0)),
        out_shape=jax.ShapeDtypeStruct((n_rows, N_EXPERTS), jnp.float32),
    )(x, wt)


_NC, _NS, _L = 2, 16, 16
_NW = _NC * _NS
_VCHUNKS = N_EXPERTS // _L


def _topk_body(rows_w, scores_hbm, vals_hbm, idx_hbm, s_v, val_v, idx_v):
    wid = lax.axis_index("s") * _NC + lax.axis_index("c")
    row0 = wid * rows_w
    pltpu.sync_copy(scores_hbm.at[pl.ds(row0 * N_EXPERTS, rows_w * N_EXPERTS)],
                    s_v)

    lanes = lax.iota(jnp.int32, _L)
    low8 = lanes < K_TOP
    idx_of = [lanes + j * _L for j in range(_VCHUNKS)]

    @plsc.parallel_loop(0, rows_w, unroll=4)
    def _(r):
        parts = []
        for j in range(_VCHUNKS):
            kj = s_v[pl.ds(r * N_EXPERTS + j * _L, _L)]
            parts.append(plsc.sort_key_val(kj, idx_of[j], descending=(j % 2 == 0)))
        (k0, v0), (k1, v1), (k2, v2), (k3, v3) = parts
        t01 = plsc.sort_key_val(jnp.where(low8, k0, k1), jnp.where(low8, v0, v1),
                                descending=True)
        t23 = plsc.sort_key_val(jnp.where(low8, k2, k3), jnp.where(low8, v2, v3),
                                descending=False)
        kt, vt = plsc.sort_key_val(jnp.where(low8, t01[0], t23[0]),
                                   jnp.where(low8, t01[1], t23[1]),
                                   descending=True)
        plsc.store_compressed(val_v.at[pl.ds(r * K_TOP, _L)], kt, mask=low8)
        plsc.store_compressed(idx_v.at[pl.ds(r * K_TOP, _L)], vt, mask=low8)

    n_out = rows_w * K_TOP
    pltpu.sync_copy(val_v.at[pl.ds(0, n_out)],
                    vals_hbm.at[pl.ds(row0 * K_TOP, n_out)])
    pltpu.sync_copy(idx_v.at[pl.ds(0, n_out)],
                    idx_hbm.at[pl.ds(row0 * K_TOP, n_out)])


@functools.cache
def _get_topk(n_rows):
    rows_w = n_rows // _NW
    return pl.kernel(
        functools.partial(_topk_body, rows_w),
        out_type=[
            jax.ShapeDtypeStruct((n_rows * K_TOP,), jnp.float32),
            jax.ShapeDtypeStruct((n_rows * K_TOP,), jnp.int32),
        ],
        mesh=plsc.VectorSubcoreMesh(core_axis_name="c", subcore_axis_name="s",
                                    num_cores=_NC, num_subcores=_NS),
        compiler_params=pltpu.CompilerParams(needs_layout_passes=False),
        scratch_types=[
            pltpu.VMEM((rows_w * N_EXPERTS,), jnp.float32),
            pltpu.VMEM((rows_w * K_TOP + _L,), jnp.float32),
            pltpu.VMEM((rows_w * K_TOP + _L,), jnp.int32),
        ],
    )


def kernel(x, W):
    wt = W
    vals, idx = [], []
    row0 = 0
    for n_rows in _CHUNK_SIZES:
        scores = _gate_scores(x, wt, row0, n_rows)
        v, i = _get_topk(n_rows)(scores.reshape(-1))
        vals.append(v.reshape(n_rows, K_TOP))
        idx.append(i.reshape(n_rows, K_TOP))
        row0 += n_rows
    return (jnp.concatenate(vals, axis=0), jnp.concatenate(idx, axis=0))
